# Initial kernel scaffold; baseline (speedup 1.0000x reference)
import jax
import jax.numpy as jnp
from jax.experimental import pallas as pl


def kernel(x, edge_index, edge_attr, params):
    raise NotImplementedError("write your pallas kernel here")


# speedup vs baseline: 1.1223x; 1.1223x over previous
import functools

import jax
import jax.numpy as jnp
from jax.experimental import pallas as pl
from jax.experimental.pallas import tpu as pltpu

N_ADJ_BM = 256


def _adj_mean_body(z_ref, zt_ref, mean_ref):
    g = jnp.dot(z_ref[...], zt_ref[...], preferred_element_type=jnp.float32)
    mean_ref[...] = (jnp.sum(g, axis=1) / jnp.float32(g.shape[1]))[None, :]


def _adj_body(z_ref, zt_ref, mean_ref, out_ref):
    g = jnp.dot(z_ref[...], zt_ref[...], preferred_element_type=jnp.float32)
    out_ref[...] = jnp.tanh(g - mean_ref[...])


def _adj_head(z):
    n = z.shape[0]
    zt = z.T
    grid = (pl.cdiv(n, N_ADJ_BM),)
    means = pl.pallas_call(
        _adj_mean_body,
        grid=grid,
        in_specs=[
            pl.BlockSpec((N_ADJ_BM, 2), lambda i: (i, 0)),
            pl.BlockSpec((2, n), lambda i: (0, 0)),
        ],
        out_specs=pl.BlockSpec((1, N_ADJ_BM), lambda i: (0, i)),
        out_shape=jax.ShapeDtypeStruct((1, n), jnp.float32),
    )(z, zt)
    return pl.pallas_call(
        _adj_body,
        grid=grid,
        in_specs=[
            pl.BlockSpec((N_ADJ_BM, 2), lambda i: (i, 0)),
            pl.BlockSpec((2, n), lambda i: (0, 0)),
            pl.BlockSpec((1, n), lambda i: (0, 0)),
        ],
        out_specs=pl.BlockSpec((N_ADJ_BM, n), lambda i: (i, 0)),
        out_shape=jax.ShapeDtypeStruct((n, n), jnp.float32),
    )(z, zt, means)


def _gat_layer(x, src, dst, edge_attr, p, num_nodes):
    hl = x @ p['Wl']
    hr = x @ p['Wr']
    he = edge_attr @ p['We']
    e = jax.nn.leaky_relu(hl[src] + hr[dst] + he, negative_slope=0.2)
    score = e @ p['att']
    smax = jax.ops.segment_max(score, dst, num_segments=num_nodes)
    ex = jnp.exp(score - smax[dst])
    den = jax.ops.segment_sum(ex, dst, num_segments=num_nodes)
    alpha = ex / (den[dst] + 1e-16)
    return jax.ops.segment_sum(alpha[:, None] * hl[src], dst, num_segments=num_nodes)


def kernel(x, edge_index, edge_attr, params):
    src = edge_index[0]
    dst = edge_index[1]
    n = x.shape[0]
    h = jax.nn.relu(_gat_layer(x, src, dst, edge_attr, params['enc1'], n))
    z = _gat_layer(h, src, dst, edge_attr, params['enc2'], n)
    d = jax.nn.relu(_gat_layer(z, src, dst, edge_attr, params['dec1'], n))
    x_recon = _gat_layer(d, src, dst, edge_attr, params['dec2'], n)
    ef = jnp.concatenate([z[src], z[dst]], axis=1)
    hmid = jax.nn.relu(ef @ params['mlp']['W1'] + params['mlp']['b1'])
    edge_recon = hmid @ params['mlp']['W2'] + params['mlp']['b2']
    adj = _adj_head(z)
    return x_recon, edge_recon, adj


# speedup vs baseline: 7.7316x; 6.8894x over previous
import functools

import jax
import jax.numpy as jnp
from jax import lax
from jax.experimental import pallas as pl
from jax.experimental.pallas import tpu as pltpu
from jax.experimental.pallas import tpu_sc as plsc

N_NODES = 10000
N_EDGES = 160000
CHUNK = 128
N_WORKERS = 32
CH_PER_W = 40
E_PAD = N_WORKERS * CH_PER_W * CHUNK
ACC_ROWS = 10112
ROWS_PER_TILE = ACC_ROWS // 16
W_ACC = 16

N_ADJ_BM = 256
N_MLP_BM = 4000

_SC_PARAMS = pltpu.CompilerParams(use_tc_tiling_on_sc=False)


@functools.lru_cache(maxsize=None)
def _gat_edge_sc(unused_f):
    mesh = plsc.VectorSubcoreMesh(core_axis_name="c", subcore_axis_name="s")

    @functools.partial(
        pl.kernel,
        out_type=jax.ShapeDtypeStruct((2, ACC_ROWS, W_ACC), jnp.float32),
        mesh=mesh,
        compiler_params=_SC_PARAMS,
        scratch_types=[
            pltpu.VMEM((CHUNK,), jnp.int32),
            pltpu.VMEM((CHUNK,), jnp.int32),
            pltpu.VMEM((CHUNK, 16), jnp.float32),
            pltpu.VMEM((CHUNK, 16), jnp.float32),
            pltpu.VMEM((CHUNK, 16), jnp.float32),
            pltpu.VMEM((CHUNK, 16), jnp.float32),
            pltpu.VMEM((ROWS_PER_TILE, 16), jnp.float32),
            pltpu.VMEM((16,), jnp.float32),
            pltpu.VMEM_SHARED((ACC_ROWS, W_ACC), jnp.float32),
            pltpu.SemaphoreType.DMA,
            pltpu.SemaphoreType.DMA,
        ],
    )
    def k(hl, hr, he, src, dst, att, out,
          src_v, dst_v, a_v, b_v, c_v, vals_v, bounce_v, att_v, accum,
          sem_a, sem_b):
        c = lax.axis_index("c")
        s = lax.axis_index("s")
        wid = c * 16 + s
        zeros16 = jnp.zeros((16,), jnp.float32)

        @pl.loop(0, ROWS_PER_TILE)
        def _zb(i):
            bounce_v[i, :] = zeros16

        row0 = s * ROWS_PER_TILE
        pltpu.sync_copy(bounce_v, accum.at[pl.ds(row0, ROWS_PER_TILE)])
        plsc.subcore_barrier()

        pltpu.sync_copy(att, att_v)
        attv = att_v[...]
        base0 = wid * (CH_PER_W * CHUNK)

        @pl.loop(0, CH_PER_W)
        def _chunk(j):
            base = base0 + j * CHUNK
            pltpu.sync_copy(src.at[pl.ds(base, CHUNK)], src_v)
            pltpu.sync_copy(dst.at[pl.ds(base, CHUNK)], dst_v)
            ca = pltpu.async_copy(hl.at[src_v], a_v, sem_a)
            cb = pltpu.async_copy(hr.at[dst_v], b_v, sem_b)
            pltpu.sync_copy(he.at[pl.ds(base, CHUNK)], c_v)
            ca.wait()
            cb.wait()

            @pl.loop(0, CHUNK, unroll=16)
            def _e(e):
                ar = a_v[e, :]
                sg = ar + b_v[e, :] + c_v[e, :]
                lr = jnp.maximum(sg, jnp.float32(0.2) * sg)
                u = attv * lr
                w = u + lax.rev(u, (0,))
                sc = ((w[0] + w[1]) + (w[2] + w[3])
                      + (w[4] + w[5]) + (w[6] + w[7]))
                ex = jnp.exp(sc + zeros16)
                vals_v[e, :] = ex * ar

            pltpu.sync_copy(vals_v, accum.at[dst_v], add=True)

        plsc.subcore_barrier()
        pltpu.sync_copy(accum.at[pl.ds(row0, ROWS_PER_TILE)], bounce_v)
        pltpu.sync_copy(bounce_v, out.at[c, pl.ds(row0, ROWS_PER_TILE)])

    return k


@functools.lru_cache(maxsize=None)
def _mlp_edge_sc():
    mesh = plsc.VectorSubcoreMesh(core_axis_name="c", subcore_axis_name="s")

    @functools.partial(
        pl.kernel,
        out_type=jax.ShapeDtypeStruct((E_PAD, 16), jnp.float32),
        mesh=mesh,
        compiler_params=_SC_PARAMS,
        scratch_types=[
            pltpu.VMEM((CHUNK,), jnp.int32),
            pltpu.VMEM((CHUNK,), jnp.int32),
            pltpu.VMEM((CHUNK, 16), jnp.float32),
            pltpu.VMEM((CHUNK, 16), jnp.float32),
            pltpu.VMEM((CHUNK, 16), jnp.float32),
            pltpu.VMEM((16,), jnp.float32),
            pltpu.SemaphoreType.DMA,
            pltpu.SemaphoreType.DMA,
        ],
    )
    def k(zp, zq, src, dst, b1, out,
          src_v, dst_v, a_v, b_v, vals_v, b1_v, sem_a, sem_b):
        c = lax.axis_index("c")
        s = lax.axis_index("s")
        wid = c * 16 + s
        zeros16 = jnp.zeros((16,), jnp.float32)

        pltpu.sync_copy(b1, b1_v)
        b1v = b1_v[...]
        base0 = wid * (CH_PER_W * CHUNK)

        @pl.loop(0, CH_PER_W)
        def _chunk(j):
            base = base0 + j * CHUNK
            pltpu.sync_copy(src.at[pl.ds(base, CHUNK)], src_v)
            pltpu.sync_copy(dst.at[pl.ds(base, CHUNK)], dst_v)
            ca = pltpu.async_copy(zp.at[src_v], a_v, sem_a)
            cb = pltpu.async_copy(zq.at[dst_v], b_v, sem_b)
            ca.wait()
            cb.wait()

            @pl.loop(0, CHUNK, unroll=16)
            def _e(e):
                vals_v[e, :] = jnp.maximum(a_v[e, :] + b_v[e, :] + b1v,
                                           zeros16)

            pltpu.sync_copy(vals_v, out.at[pl.ds(base, CHUNK)])

    return k


def _padded(h, marker):
    n, fo = h.shape
    cols = [h, jnp.full((n, 1), marker, jnp.float32)]
    if fo < 15:
        cols.append(jnp.zeros((n, 15 - fo), jnp.float32))
    return jnp.concatenate(cols, axis=1)


def _nodes_body(x_ref, wl_ref, wr_ref, hl_ref, hr_ref):
    x = x_ref[...]
    hl_ref[...] = _padded(
        jnp.dot(x, wl_ref[...], preferred_element_type=jnp.float32), 1.0)
    hr_ref[...] = _padded(
        jnp.dot(x, wr_ref[...], preferred_element_type=jnp.float32), 0.0)


def _node_proj(x, wl, wr):
    n, fin = x.shape
    fo = wl.shape[1]
    return pl.pallas_call(
        _nodes_body,
        in_specs=[pl.BlockSpec((n, fin), lambda: (0, 0)),
                  pl.BlockSpec((fin, fo), lambda: (0, 0)),
                  pl.BlockSpec((fin, fo), lambda: (0, 0))],
        out_specs=[pl.BlockSpec((n, 16), lambda: (0, 0)),
                   pl.BlockSpec((n, 16), lambda: (0, 0))],
        out_shape=[jax.ShapeDtypeStruct((n, 16), jnp.float32),
                   jax.ShapeDtypeStruct((n, 16), jnp.float32)],
    )(x, wl, wr)


def _he_body(ea_ref, w_ref, o1_ref, o2_ref, o3_ref, o4_ref):
    h = jnp.dot(ea_ref[...], w_ref[...], preferred_element_type=jnp.float32)
    o1_ref[...] = _padded(h[:, 0:8], 0.0)
    o2_ref[...] = _padded(h[:, 8:10], 0.0)
    o3_ref[...] = _padded(h[:, 10:18], 0.0)
    o4_ref[...] = _padded(h[:, 18:33], 0.0)


def _he_proj(ea_pad, wecat):
    bm = 4096
    grid = (E_PAD // bm,)
    return pl.pallas_call(
        _he_body,
        grid=grid,
        in_specs=[pl.BlockSpec((bm, 35), lambda i: (i, 0)),
                  pl.BlockSpec((35, 33), lambda i: (0, 0))],
        out_specs=[pl.BlockSpec((bm, 16), lambda i: (i, 0)),
                   pl.BlockSpec((bm, 16), lambda i: (i, 0)),
                   pl.BlockSpec((bm, 16), lambda i: (i, 0)),
                   pl.BlockSpec((bm, 16), lambda i: (i, 0))],
        out_shape=[jax.ShapeDtypeStruct((E_PAD, 16), jnp.float32)
                   for _ in range(4)],
    )(ea_pad, wecat)


def _combine(acc, f, act, weights, markers):
    n = N_NODES

    def body(acc_ref, *rest):
        w_refs = rest[:len(weights)]
        o_refs = rest[len(weights):]
        num = acc_ref[0, :n, 0:f] + acc_ref[1, :n, 0:f]
        den = acc_ref[0, :n, f:f + 1] + acc_ref[1, :n, f:f + 1]
        h = num / (den + jnp.float32(1e-16))
        if act:
            h = jnp.maximum(h, jnp.float32(0.0))
        o_refs[0][...] = h
        for w_ref, o_ref, m in zip(w_refs, o_refs[1:], markers):
            o_ref[...] = _padded(
                jnp.dot(h, w_ref[...], preferred_element_type=jnp.float32), m)

    out_shapes = [jax.ShapeDtypeStruct((n, f), jnp.float32)]
    in_specs = [pl.BlockSpec((2, ACC_ROWS, W_ACC), lambda: (0, 0, 0))]
    for w in weights:
        in_specs.append(pl.BlockSpec(w.shape, lambda: (0, 0)))
        out_shapes.append(jax.ShapeDtypeStruct((n, 16), jnp.float32))
    out_specs = [pl.BlockSpec(o.shape, lambda: (0, 0)) for o in out_shapes]
    return pl.pallas_call(
        body,
        in_specs=in_specs,
        out_specs=out_specs,
        out_shape=out_shapes,
    )(acc, *weights)


def _mlp_out_body(hmid_ref, w2_ref, b2_ref, out_ref):
    h = hmid_ref[...][:, 0:15]
    out_ref[...] = (jnp.dot(h, w2_ref[...], preferred_element_type=jnp.float32)
                    + b2_ref[...])


def _mlp_out(hmid, w2, b2):
    grid = (N_EDGES // N_MLP_BM,)
    return pl.pallas_call(
        _mlp_out_body,
        grid=grid,
        in_specs=[pl.BlockSpec((N_MLP_BM, 16), lambda i: (i, 0)),
                  pl.BlockSpec((15, 35), lambda i: (0, 0)),
                  pl.BlockSpec((1, 35), lambda i: (0, 0))],
        out_specs=pl.BlockSpec((N_MLP_BM, 35), lambda i: (i, 0)),
        out_shape=jax.ShapeDtypeStruct((N_EDGES, 35), jnp.float32),
    )(hmid, w2, b2)


def _adj_mean_body(z_ref, zt_ref, mean_ref):
    g = jnp.dot(z_ref[...], zt_ref[...], preferred_element_type=jnp.float32)
    mean_ref[...] = (jnp.sum(g, axis=1) / jnp.float32(g.shape[1]))[None, :]


def _adj_body(z_ref, zt_ref, mean_ref, out_ref):
    g = jnp.dot(z_ref[...], zt_ref[...], preferred_element_type=jnp.float32)
    out_ref[...] = jnp.tanh(g - mean_ref[...])


def _adj_head(z):
    n = z.shape[0]
    zt = z.T
    grid = (pl.cdiv(n, N_ADJ_BM),)
    means = pl.pallas_call(
        _adj_mean_body,
        grid=grid,
        in_specs=[
            pl.BlockSpec((N_ADJ_BM, 2), lambda i: (i, 0)),
            pl.BlockSpec((2, n), lambda i: (0, 0)),
        ],
        out_specs=pl.BlockSpec((1, N_ADJ_BM), lambda i: (0, i)),
        out_shape=jax.ShapeDtypeStruct((1, n), jnp.float32),
    )(z, zt)
    return pl.pallas_call(
        _adj_body,
        grid=grid,
        in_specs=[
            pl.BlockSpec((N_ADJ_BM, 2), lambda i: (i, 0)),
            pl.BlockSpec((2, n), lambda i: (0, 0)),
            pl.BlockSpec((1, n), lambda i: (0, 0)),
        ],
        out_specs=pl.BlockSpec((N_ADJ_BM, n), lambda i: (i, 0)),
        out_shape=jax.ShapeDtypeStruct((n, n), jnp.float32),
    )(z, zt, means)


def _pad16(v):
    return jnp.pad(v, (0, 16 - v.shape[0]))


def kernel(x, edge_index, edge_attr, params):
    src = edge_index[0]
    dst = edge_index[1]
    npad = E_PAD - N_EDGES
    src_pad = jnp.concatenate([src, jnp.zeros((npad,), jnp.int32)])
    dst_pad = jnp.concatenate([dst, jnp.full((npad,), N_NODES, jnp.int32)])
    ea_pad = jnp.pad(edge_attr, ((0, npad), (0, 0)))

    p1, p2, p3, p4 = (params['enc1'], params['enc2'],
                      params['dec1'], params['dec2'])
    mlp = params['mlp']
    wecat = jnp.concatenate([p1['We'], p2['We'], p3['We'], p4['We']], axis=1)
    he1, he2, he3, he4 = _he_proj(ea_pad, wecat)

    hl1, hr1 = _node_proj(x, p1['Wl'], p1['Wr'])
    acc1 = _gat_edge_sc(8)(hl1, hr1, he1, src_pad, dst_pad, _pad16(p1['att']))
    hl2, hr2 = _combine(acc1, 8, True, (p2['Wl'], p2['Wr']), (1.0, 0.0))[1:]
    acc2 = _gat_edge_sc(2)(hl2, hr2, he2, src_pad, dst_pad, _pad16(p2['att']))
    z, hl3, hr3, zp, zq = _combine(
        acc2, 2, False,
        (p3['Wl'], p3['Wr'], mlp['W1'][0:2], mlp['W1'][2:4]),
        (1.0, 0.0, 0.0, 0.0))
    acc3 = _gat_edge_sc(8)(hl3, hr3, he3, src_pad, dst_pad, _pad16(p3['att']))
    hl4, hr4 = _combine(acc3, 8, True, (p4['Wl'], p4['Wr']), (1.0, 0.0))[1:]
    acc4 = _gat_edge_sc(15)(hl4, hr4, he4, src_pad, dst_pad, _pad16(p4['att']))
    x_recon = _combine(acc4, 15, False, (), ())[0]

    hmid = _mlp_edge_sc()(zp, zq, src_pad, dst_pad, _pad16(mlp['b1']))
    edge_recon = _mlp_out(hmid, mlp['W2'], mlp['b2'][None, :])
    adj = _adj_head(z)
    return x_recon, edge_recon, adj


# speedup vs baseline: 9.5036x; 1.2292x over previous
import functools

import jax
import jax.numpy as jnp
from jax import lax
from jax.experimental import pallas as pl
from jax.experimental.pallas import tpu as pltpu
from jax.experimental.pallas import tpu_sc as plsc

N_NODES = 10000
N_EDGES = 160000
CHUNK = 128
N_WORKERS = 32
CH_PER_W = 40
E_PAD = N_WORKERS * CH_PER_W * CHUNK
ACC_ROWS = 10112
ROWS_PER_TILE = ACC_ROWS // 16
W_ACC = 16

N_ADJ_BM = 256
N_MLP_BM = 4000

_SC_PARAMS = pltpu.CompilerParams(use_tc_tiling_on_sc=False)


@functools.lru_cache(maxsize=None)
def _gat_edge_sc(unused_f):
    mesh = plsc.VectorSubcoreMesh(core_axis_name="c", subcore_axis_name="s")

    @functools.partial(
        pl.kernel,
        out_type=jax.ShapeDtypeStruct((2, ACC_ROWS, W_ACC), jnp.float32),
        mesh=mesh,
        compiler_params=_SC_PARAMS,
        scratch_types=[
            pltpu.VMEM((CH_PER_W, CHUNK), jnp.int32),
            pltpu.VMEM((CH_PER_W, CHUNK), jnp.int32),
            pltpu.VMEM((2, CHUNK, 16), jnp.float32),
            pltpu.VMEM((2, CHUNK, 16), jnp.float32),
            pltpu.VMEM((2, CHUNK, 16), jnp.float32),
            pltpu.VMEM((2, CHUNK, 16), jnp.float32),
            pltpu.VMEM((ROWS_PER_TILE, 16), jnp.float32),
            pltpu.VMEM((16,), jnp.float32),
            pltpu.VMEM_SHARED((ACC_ROWS, W_ACC), jnp.float32),
            pltpu.SemaphoreType.DMA((2,)),
            pltpu.SemaphoreType.DMA((2,)),
        ],
    )
    def k(hl, hr, he, src2d, dst2d, att, out,
          src_v, dst_v, a_v, b_v, c_v, vals_v, bounce_v, att_v, accum,
          sem_ld, sem_sc):
        c = lax.axis_index("c")
        s = lax.axis_index("s")
        wid = c * 16 + s
        zeros16 = jnp.zeros((16,), jnp.float32)

        @pl.loop(0, ROWS_PER_TILE)
        def _zb(i):
            bounce_v[i, :] = zeros16

        row0 = s * ROWS_PER_TILE
        pltpu.sync_copy(bounce_v, accum.at[pl.ds(row0, ROWS_PER_TILE)])
        plsc.subcore_barrier()

        pltpu.sync_copy(att, att_v)
        attv = att_v[...]
        ch0 = wid * CH_PER_W
        pltpu.sync_copy(src2d.at[pl.ds(ch0, CH_PER_W)], src_v)
        pltpu.sync_copy(dst2d.at[pl.ds(ch0, CH_PER_W)], dst_v)
        base0 = ch0 * CHUNK

        def fire_loads(j):
            b = j & 1
            pltpu.async_copy(hl.at[src_v.at[j]], a_v.at[b], sem_ld.at[b])
            pltpu.async_copy(hr.at[dst_v.at[j]], b_v.at[b], sem_ld.at[b])
            pltpu.async_copy(he.at[pl.ds(base0 + j * CHUNK, CHUNK)],
                             c_v.at[b], sem_ld.at[b])

        def wait_loads(j):
            b = j & 1
            pltpu.make_async_copy(hl.at[src_v.at[j]], a_v.at[b],
                                  sem_ld.at[b]).wait()
            pltpu.make_async_copy(hr.at[dst_v.at[j]], b_v.at[b],
                                  sem_ld.at[b]).wait()
            pltpu.make_async_copy(he.at[pl.ds(base0 + j * CHUNK, CHUNK)],
                                  c_v.at[b], sem_ld.at[b]).wait()

        def wait_scatter(j):
            b = j & 1
            pltpu.make_async_copy(vals_v.at[b], accum.at[dst_v.at[j]],
                                  sem_sc.at[b]).wait()

        fire_loads(0)

        @pl.loop(0, CH_PER_W)
        def _chunk(j):
            b = j & 1

            @pl.when(j + 1 < CH_PER_W)
            def _():
                fire_loads(j + 1)

            wait_loads(j)

            @pl.when(j >= 2)
            def _():
                wait_scatter(j - 2)

            @pl.loop(0, CHUNK, unroll=16)
            def _e(e):
                ar = a_v[b, e, :]
                sg = ar + b_v[b, e, :] + c_v[b, e, :]
                lr = jnp.maximum(sg, jnp.float32(0.2) * sg)
                u = attv * lr
                w = u + lax.rev(u, (0,))
                sc = ((w[0] + w[1]) + (w[2] + w[3])
                      + (w[4] + w[5]) + (w[6] + w[7]))
                ex = jnp.exp(sc + zeros16)
                vals_v[b, e, :] = ex * ar

            pltpu.async_copy(vals_v.at[b], accum.at[dst_v.at[j]],
                             sem_sc.at[b], add=True)

        wait_scatter(CH_PER_W - 2)
        wait_scatter(CH_PER_W - 1)
        plsc.subcore_barrier()
        pltpu.sync_copy(accum.at[pl.ds(row0, ROWS_PER_TILE)], bounce_v)
        pltpu.sync_copy(bounce_v, out.at[c, pl.ds(row0, ROWS_PER_TILE)])

    return k


@functools.lru_cache(maxsize=None)
def _mlp_edge_sc():
    mesh = plsc.VectorSubcoreMesh(core_axis_name="c", subcore_axis_name="s")

    @functools.partial(
        pl.kernel,
        out_type=jax.ShapeDtypeStruct((E_PAD, 16), jnp.float32),
        mesh=mesh,
        compiler_params=_SC_PARAMS,
        scratch_types=[
            pltpu.VMEM((CH_PER_W, CHUNK), jnp.int32),
            pltpu.VMEM((CH_PER_W, CHUNK), jnp.int32),
            pltpu.VMEM((2, CHUNK, 16), jnp.float32),
            pltpu.VMEM((2, CHUNK, 16), jnp.float32),
            pltpu.VMEM((2, CHUNK, 16), jnp.float32),
            pltpu.VMEM((16,), jnp.float32),
            pltpu.SemaphoreType.DMA((2,)),
            pltpu.SemaphoreType.DMA((2,)),
        ],
    )
    def k(zp, zq, src2d, dst2d, b1, out,
          src_v, dst_v, a_v, b_v, vals_v, b1_v, sem_ld, sem_st):
        c = lax.axis_index("c")
        s = lax.axis_index("s")
        wid = c * 16 + s
        zeros16 = jnp.zeros((16,), jnp.float32)

        pltpu.sync_copy(b1, b1_v)
        b1v = b1_v[...]
        ch0 = wid * CH_PER_W
        pltpu.sync_copy(src2d.at[pl.ds(ch0, CH_PER_W)], src_v)
        pltpu.sync_copy(dst2d.at[pl.ds(ch0, CH_PER_W)], dst_v)
        base0 = ch0 * CHUNK

        def fire_loads(j):
            b = j & 1
            pltpu.async_copy(zp.at[src_v.at[j]], a_v.at[b], sem_ld.at[b])
            pltpu.async_copy(zq.at[dst_v.at[j]], b_v.at[b], sem_ld.at[b])

        def wait_loads(j):
            b = j & 1
            pltpu.make_async_copy(zp.at[src_v.at[j]], a_v.at[b],
                                  sem_ld.at[b]).wait()
            pltpu.make_async_copy(zq.at[dst_v.at[j]], b_v.at[b],
                                  sem_ld.at[b]).wait()

        def wait_store(j):
            b = j & 1
            pltpu.make_async_copy(
                vals_v.at[b], out.at[pl.ds(base0 + j * CHUNK, CHUNK)],
                sem_st.at[b]).wait()

        fire_loads(0)

        @pl.loop(0, CH_PER_W)
        def _chunk(j):
            b = j & 1

            @pl.when(j + 1 < CH_PER_W)
            def _():
                fire_loads(j + 1)

            wait_loads(j)

            @pl.when(j >= 2)
            def _():
                wait_store(j - 2)

            @pl.loop(0, CHUNK, unroll=16)
            def _e(e):
                vals_v[b, e, :] = jnp.maximum(
                    a_v[b, e, :] + b_v[b, e, :] + b1v, zeros16)

            pltpu.async_copy(vals_v.at[b],
                             out.at[pl.ds(base0 + j * CHUNK, CHUNK)],
                             sem_st.at[b])

        wait_store(CH_PER_W - 2)
        wait_store(CH_PER_W - 1)

    return k


def _padded(h, marker):
    n, fo = h.shape
    cols = [h, jnp.full((n, 1), marker, jnp.float32)]
    if fo < 15:
        cols.append(jnp.zeros((n, 15 - fo), jnp.float32))
    return jnp.concatenate(cols, axis=1)


def _nodes_body(x_ref, wl_ref, wr_ref, hl_ref, hr_ref):
    x = x_ref[...]
    hl_ref[...] = _padded(
        jnp.dot(x, wl_ref[...], preferred_element_type=jnp.float32), 1.0)
    hr_ref[...] = _padded(
        jnp.dot(x, wr_ref[...], preferred_element_type=jnp.float32), 0.0)


def _node_proj(x, wl, wr):
    n, fin = x.shape
    fo = wl.shape[1]
    return pl.pallas_call(
        _nodes_body,
        in_specs=[pl.BlockSpec((n, fin), lambda: (0, 0)),
                  pl.BlockSpec((fin, fo), lambda: (0, 0)),
                  pl.BlockSpec((fin, fo), lambda: (0, 0))],
        out_specs=[pl.BlockSpec((n, 16), lambda: (0, 0)),
                   pl.BlockSpec((n, 16), lambda: (0, 0))],
        out_shape=[jax.ShapeDtypeStruct((n, 16), jnp.float32),
                   jax.ShapeDtypeStruct((n, 16), jnp.float32)],
    )(x, wl, wr)


def _he_body(ea_ref, w_ref, o1_ref, o2_ref, o3_ref, o4_ref):
    h = jnp.dot(ea_ref[...], w_ref[...], preferred_element_type=jnp.float32)
    o1_ref[...] = _padded(h[:, 0:8], 0.0)
    o2_ref[...] = _padded(h[:, 8:10], 0.0)
    o3_ref[...] = _padded(h[:, 10:18], 0.0)
    o4_ref[...] = _padded(h[:, 18:33], 0.0)


def _he_proj(ea_pad, wecat):
    bm = 4096
    grid = (E_PAD // bm,)
    return pl.pallas_call(
        _he_body,
        grid=grid,
        in_specs=[pl.BlockSpec((bm, 35), lambda i: (i, 0)),
                  pl.BlockSpec((35, 33), lambda i: (0, 0))],
        out_specs=[pl.BlockSpec((bm, 16), lambda i: (i, 0)),
                   pl.BlockSpec((bm, 16), lambda i: (i, 0)),
                   pl.BlockSpec((bm, 16), lambda i: (i, 0)),
                   pl.BlockSpec((bm, 16), lambda i: (i, 0))],
        out_shape=[jax.ShapeDtypeStruct((E_PAD, 16), jnp.float32)
                   for _ in range(4)],
    )(ea_pad, wecat)


def _combine(acc, f, act, weights, markers):
    n = N_NODES

    def body(acc_ref, *rest):
        w_refs = rest[:len(weights)]
        o_refs = rest[len(weights):]
        num = acc_ref[0, :n, 0:f] + acc_ref[1, :n, 0:f]
        den = acc_ref[0, :n, f:f + 1] + acc_ref[1, :n, f:f + 1]
        h = num / (den + jnp.float32(1e-16))
        if act:
            h = jnp.maximum(h, jnp.float32(0.0))
        o_refs[0][...] = h
        for w_ref, o_ref, m in zip(w_refs, o_refs[1:], markers):
            o_ref[...] = _padded(
                jnp.dot(h, w_ref[...], preferred_element_type=jnp.float32), m)

    out_shapes = [jax.ShapeDtypeStruct((n, f), jnp.float32)]
    in_specs = [pl.BlockSpec((2, ACC_ROWS, W_ACC), lambda: (0, 0, 0))]
    for w in weights:
        in_specs.append(pl.BlockSpec(w.shape, lambda: (0, 0)))
        out_shapes.append(jax.ShapeDtypeStruct((n, 16), jnp.float32))
    out_specs = [pl.BlockSpec(o.shape, lambda: (0, 0)) for o in out_shapes]
    return pl.pallas_call(
        body,
        in_specs=in_specs,
        out_specs=out_specs,
        out_shape=out_shapes,
    )(acc, *weights)


def _mlp_out_body(hmid_ref, w2_ref, b2_ref, out_ref):
    h = hmid_ref[...][:, 0:15]
    out_ref[...] = (jnp.dot(h, w2_ref[...], preferred_element_type=jnp.float32)
                    + b2_ref[...])


def _mlp_out(hmid, w2, b2):
    grid = (N_EDGES // N_MLP_BM,)
    return pl.pallas_call(
        _mlp_out_body,
        grid=grid,
        in_specs=[pl.BlockSpec((N_MLP_BM, 16), lambda i: (i, 0)),
                  pl.BlockSpec((15, 35), lambda i: (0, 0)),
                  pl.BlockSpec((1, 35), lambda i: (0, 0))],
        out_specs=pl.BlockSpec((N_MLP_BM, 35), lambda i: (i, 0)),
        out_shape=jax.ShapeDtypeStruct((N_EDGES, 35), jnp.float32),
    )(hmid, w2, b2)


def _adj_mean_body(z_ref, zt_ref, mean_ref):
    g = jnp.dot(z_ref[...], zt_ref[...], preferred_element_type=jnp.float32)
    mean_ref[...] = (jnp.sum(g, axis=1) / jnp.float32(g.shape[1]))[None, :]


def _adj_body(z_ref, zt_ref, mean_ref, out_ref):
    g = jnp.dot(z_ref[...], zt_ref[...], preferred_element_type=jnp.float32)
    out_ref[...] = jnp.tanh(g - mean_ref[...])


def _adj_head(z):
    n = z.shape[0]
    zt = z.T
    grid = (pl.cdiv(n, N_ADJ_BM),)
    means = pl.pallas_call(
        _adj_mean_body,
        grid=grid,
        in_specs=[
            pl.BlockSpec((N_ADJ_BM, 2), lambda i: (i, 0)),
            pl.BlockSpec((2, n), lambda i: (0, 0)),
        ],
        out_specs=pl.BlockSpec((1, N_ADJ_BM), lambda i: (0, i)),
        out_shape=jax.ShapeDtypeStruct((1, n), jnp.float32),
    )(z, zt)
    return pl.pallas_call(
        _adj_body,
        grid=grid,
        in_specs=[
            pl.BlockSpec((N_ADJ_BM, 2), lambda i: (i, 0)),
            pl.BlockSpec((2, n), lambda i: (0, 0)),
            pl.BlockSpec((1, n), lambda i: (0, 0)),
        ],
        out_specs=pl.BlockSpec((N_ADJ_BM, n), lambda i: (i, 0)),
        out_shape=jax.ShapeDtypeStruct((n, n), jnp.float32),
    )(z, zt, means)


def _pad16(v):
    return jnp.pad(v, (0, 16 - v.shape[0]))


def kernel(x, edge_index, edge_attr, params):
    src = edge_index[0]
    dst = edge_index[1]
    npad = E_PAD - N_EDGES
    src_pad = jnp.concatenate(
        [src, jnp.zeros((npad,), jnp.int32)]).reshape(-1, CHUNK)
    dst_pad = jnp.concatenate(
        [dst, jnp.full((npad,), N_NODES, jnp.int32)]).reshape(-1, CHUNK)
    ea_pad = jnp.pad(edge_attr, ((0, npad), (0, 0)))

    p1, p2, p3, p4 = (params['enc1'], params['enc2'],
                      params['dec1'], params['dec2'])
    mlp = params['mlp']
    wecat = jnp.concatenate([p1['We'], p2['We'], p3['We'], p4['We']], axis=1)
    he1, he2, he3, he4 = _he_proj(ea_pad, wecat)

    hl1, hr1 = _node_proj(x, p1['Wl'], p1['Wr'])
    acc1 = _gat_edge_sc(8)(hl1, hr1, he1, src_pad, dst_pad, _pad16(p1['att']))
    hl2, hr2 = _combine(acc1, 8, True, (p2['Wl'], p2['Wr']), (1.0, 0.0))[1:]
    acc2 = _gat_edge_sc(2)(hl2, hr2, he2, src_pad, dst_pad, _pad16(p2['att']))
    z, hl3, hr3, zp, zq = _combine(
        acc2, 2, False,
        (p3['Wl'], p3['Wr'], mlp['W1'][0:2], mlp['W1'][2:4]),
        (1.0, 0.0, 0.0, 0.0))
    acc3 = _gat_edge_sc(8)(hl3, hr3, he3, src_pad, dst_pad, _pad16(p3['att']))
    hl4, hr4 = _combine(acc3, 8, True, (p4['Wl'], p4['Wr']), (1.0, 0.0))[1:]
    acc4 = _gat_edge_sc(15)(hl4, hr4, he4, src_pad, dst_pad, _pad16(p4['att']))
    x_recon = _combine(acc4, 15, False, (), ())[0]

    hmid = _mlp_edge_sc()(zp, zq, src_pad, dst_pad, _pad16(mlp['b1']))
    edge_recon = _mlp_out(hmid, mlp['W2'], mlp['b2'][None, :])
    adj = _adj_head(z)
    return x_recon, edge_recon, adj
